# baseline (speedup 1.0000x reference)
import functools

import jax
import jax.numpy as jnp
from jax import lax
from jax.experimental import pallas as pl
from jax.experimental.pallas import tpu as pltpu
from jax.experimental.pallas import tpu_sc as plsc

N = 10000
E = 320000
D = 128
HID = 256

NC, NS = 2, 16
NW = NC * NS
EPW = E // NW
CH = 64
NB = 4
NCHUNK = EPW // CH
TAIL = EPW - NCHUNK * CH
NR = 10240
RPT = NR // NS
HN = 10240
HPT = HN // NS

_mesh = plsc.VectorSubcoreMesh(core_axis_name="c", subcore_axis_name="s")


def _sc_body(with_deg, *refs):
    if with_deg:
        (x_hbm, src_hbm, dst_hbm, out0, out1, deg0, deg1,
         sb0, sb1, sb2, sb3, db0, db1, db2, db3,
         rw0, rw1, rw2, rw3, srct, dstt, rowst,
         onesb, sdeg, degstage, accum,
         gs0, gs1, gs2, gs3, ss0, ss1, ss2, ss3,
         is0, is1, is2, is3, tsem) = refs
    else:
        (x_hbm, src_hbm, dst_hbm, out0, out1,
         sb0, sb1, sb2, sb3, db0, db1, db2, db3,
         rw0, rw1, rw2, rw3, srct, dstt, rowst, accum,
         gs0, gs1, gs2, gs3, ss0, ss1, ss2, ss3,
         is0, is1, is2, is3, tsem) = refs
    srcb = (sb0, sb1, sb2, sb3)
    dstb = (db0, db1, db2, db3)
    rows = (rw0, rw1, rw2, rw3)
    gs = (gs0, gs1, gs2, gs3)
    ss = (ss0, ss1, ss2, ss3)
    isem = (is0, is1, is2, is3)

    cid = lax.axis_index("c")
    sid = lax.axis_index("s")
    wid = sid * NC + cid
    row0 = sid * RPT

    def _zb(i, c):
        for j in range(D // 16):
            rw0[i, pl.ds(j * 16, 16)] = jnp.zeros((16,), jnp.float32)
        return c
    lax.fori_loop(0, CH, _zb, 0)
    for k in range(RPT // CH):
        pltpu.sync_copy(rw0, accum.at[pl.ds(row0 + k * CH, CH), :])
    if with_deg:
        for i in range(CH // 16):
            onesb[pl.ds(i * 16, 16)] = jnp.ones((16,), jnp.float32)

        def _zd(i, c):
            degstage[pl.ds(i * 16, 16)] = jnp.zeros((16,), jnp.float32)
            return c
        lax.fori_loop(0, HPT // 16, _zd, 0)
        pltpu.sync_copy(degstage, sdeg.at[pl.ds(sid * HPT, HPT)])
    plsc.subcore_barrier()

    base = wid * EPW

    def _load(off, j):
        pltpu.async_copy(src_hbm.at[pl.ds(off, CH)], srcb[j], isem[j])
        pltpu.async_copy(dst_hbm.at[pl.ds(off, CH)], dstb[j], isem[j])

    def _wait_load(j):
        pltpu.make_async_copy(src_hbm.at[pl.ds(0, CH)], srcb[j], isem[j]).wait()
        pltpu.make_async_copy(dst_hbm.at[pl.ds(0, CH)], dstb[j], isem[j]).wait()

    def _gather(j):
        pltpu.async_copy(x_hbm.at[srcb[j]], rows[j], gs[j])

    def _wait_gather(j):
        pltpu.make_async_copy(x_hbm.at[srcb[j]], rows[j], gs[j]).wait()

    def _scatter(j):
        pltpu.async_copy(rows[j], accum.at[dstb[j]], ss[j], add=True)
        if with_deg:
            pltpu.async_copy(onesb, sdeg.at[dstb[j]], ss[j], add=True)

    def _wait_scatter(j):
        pltpu.make_async_copy(rows[j], accum.at[dstb[j]], ss[j]).wait()
        if with_deg:
            pltpu.make_async_copy(onesb, sdeg.at[dstb[j]], ss[j]).wait()

    _load(base, 0)
    _wait_load(0)
    _gather(0)
    _load(base + CH, 1)
    _wait_load(1)
    _gather(1)

    for b in range(NB):
        j2 = (b + 2) % NB
        if b >= 2:
            _wait_scatter(j2)
        _load(base + (b + 2) * CH, j2)
        _wait_gather(b)
        _scatter(b)
        _wait_load(j2)
        _gather(j2)

    def _block(g, c):
        for b in range(NB):
            ch = g * NB + b
            j2 = (b + 2) % NB
            _wait_scatter(j2)
            _load(base + (ch + 2) * CH, j2)
            _wait_gather(b)
            _scatter(b)
            _wait_load(j2)
            _gather(j2)
        return c
    lax.fori_loop(1, NCHUNK // NB - 1, _block, 0)

    for b in range(NB):
        ch = NCHUNK - NB + b
        j2 = (b + 2) % NB
        if b < 2:
            _wait_scatter(j2)
            _load(base + (ch + 2) * CH, j2)
        _wait_gather(b)
        _scatter(b)
        if b < 2:
            _wait_load(j2)
            _gather(j2)

    for j in range(NB):
        _wait_scatter(j)

    offt = base + NCHUNK * CH
    pltpu.sync_copy(src_hbm.at[pl.ds(offt, TAIL)], srct)
    pltpu.sync_copy(dst_hbm.at[pl.ds(offt, TAIL)], dstt)
    pltpu.async_copy(x_hbm.at[srct], rowst, tsem).wait()
    pltpu.sync_copy(rowst, accum.at[dstt], add=True)
    if with_deg:
        pltpu.sync_copy(onesb.at[pl.ds(0, TAIL)], sdeg.at[dstt], add=True)
    plsc.subcore_barrier()

    for k in range(RPT // CH):
        r0 = row0 + k * CH
        pltpu.sync_copy(accum.at[pl.ds(r0, CH), :], rw0)

        @pl.when(cid == 0)
        def _():
            pltpu.sync_copy(rw0, out0.at[pl.ds(r0, CH), :])

        @pl.when(cid == 1)
        def _():
            pltpu.sync_copy(rw0, out1.at[pl.ds(r0, CH), :])
    if with_deg:
        d0row = sid * HPT
        pltpu.sync_copy(sdeg.at[pl.ds(d0row, HPT)], degstage)

        @pl.when(cid == 0)
        def _():
            pltpu.sync_copy(degstage, deg0.at[pl.ds(d0row, HPT)])

        @pl.when(cid == 1)
        def _():
            pltpu.sync_copy(degstage, deg1.at[pl.ds(d0row, HPT)])


_f32 = jnp.float32


def _sc_scratch(with_deg):
    s = []
    s += [pltpu.VMEM((CH,), jnp.int32)] * 4
    s += [pltpu.VMEM((CH,), jnp.int32)] * 4
    s += [pltpu.VMEM((CH, D), _f32)] * 4
    s += [pltpu.VMEM((TAIL,), jnp.int32)] * 2
    s += [pltpu.VMEM((TAIL, D), _f32)]
    if with_deg:
        s += [pltpu.VMEM((CH,), _f32)]
        s += [pltpu.VMEM_SHARED((HN,), _f32)]
        s += [pltpu.VMEM((HPT,), _f32)]
    s += [pltpu.VMEM_SHARED((NR, D), _f32)]
    s += [pltpu.SemaphoreType.DMA] * 13
    return s


_sc_pass1 = pl.kernel(
    functools.partial(_sc_body, True),
    out_type=(jax.ShapeDtypeStruct((NR, D), _f32),
              jax.ShapeDtypeStruct((NR, D), _f32),
              jax.ShapeDtypeStruct((HN,), _f32),
              jax.ShapeDtypeStruct((HN,), _f32)),
    mesh=_mesh,
    scratch_types=_sc_scratch(True),
    name="sage_sc_pass1",
)

_sc_pass2 = pl.kernel(
    functools.partial(_sc_body, False),
    out_type=(jax.ShapeDtypeStruct((NR, D), _f32),
              jax.ShapeDtypeStruct((NR, D), _f32)),
    mesh=_mesh,
    scratch_types=_sc_scratch(False),
    name="sage_sc_pass2",
)

BM = 400
GRID = N // BM

_DN = (((1,), (1,)), ((), ()))


def _tc_mid_body(p0, p1, d0, d1, x, w1l, b1, w1r, w2l, b2, w2r, y2, z2):
    r = 1.0 / jnp.maximum(d0[...] + d1[...], 1.0)
    agg = (p0[...] + p1[...]) * r
    h = lax.dot_general(agg, w1l[...], _DN, preferred_element_type=_f32)
    h += lax.dot_general(x[...], w1r[...], _DN, preferred_element_type=_f32)
    h = jnp.maximum(h + b1[...], 0.0)
    y2[...] = lax.dot_general(h, w2l[...], _DN, preferred_element_type=_f32)
    z2[...] = lax.dot_general(h, w2r[...], _DN, preferred_element_type=_f32) + b2[...]


def _row_spec(c):
    return pl.BlockSpec((BM, c), lambda i: (i, 0))


def _full_spec(r, c):
    return pl.BlockSpec((r, c), lambda i: (0, 0))


_tc_mid = pl.pallas_call(
    _tc_mid_body,
    grid=(GRID,),
    in_specs=[
        _row_spec(D), _row_spec(D), _row_spec(1), _row_spec(1), _row_spec(D),
        _full_spec(HID, D), _full_spec(1, HID), _full_spec(HID, D),
        _full_spec(D, HID), _full_spec(1, D), _full_spec(D, HID),
    ],
    out_specs=[_row_spec(D), _row_spec(D)],
    out_shape=[jax.ShapeDtypeStruct((N, D), _f32),
               jax.ShapeDtypeStruct((N, D), _f32)],
    name="sage_tc_mid",
)


def _tc_out_body(p0, p1, d0, d1, z2, out):
    r = 1.0 / jnp.maximum(d0[...] + d1[...], 1.0)
    out[...] = (p0[...] + p1[...]) * r + z2[...]


_tc_out = pl.pallas_call(
    _tc_out_body,
    grid=(GRID,),
    in_specs=[_row_spec(D), _row_spec(D), _row_spec(1), _row_spec(1),
              _row_spec(D)],
    out_specs=_row_spec(D),
    out_shape=jax.ShapeDtypeStruct((N, D), _f32),
    name="sage_tc_out",
)


def kernel(x, edge_index, W1l, b1l, W1r, W2l, b2l, W2r):
    src = edge_index[0]
    dst = edge_index[1]
    p0, p1, dg0, dg1 = _sc_pass1(x, src, dst)
    d0 = dg0[:N].reshape(N, 1)
    d1 = dg1[:N].reshape(N, 1)
    y2, z2 = _tc_mid(p0, p1, d0, d1, x, W1l, b1l.reshape(1, HID), W1r,
                     W2l, b2l.reshape(1, D), W2r)
    q0, q1 = _sc_pass2(y2, src, dst)
    return _tc_out(q0, q1, d0, d1, z2)


# speedup vs baseline: 1.0145x; 1.0145x over previous
import functools

import jax
import jax.numpy as jnp
from jax import lax
from jax.experimental import pallas as pl
from jax.experimental.pallas import tpu as pltpu
from jax.experimental.pallas import tpu_sc as plsc

N = 10000
E = 320000
D = 128
HID = 256

NC, NS = 2, 16
NW = NC * NS
EPW = E // NW
CH = 64
NB = 4
NCHUNK = EPW // CH
TAIL = EPW - NCHUNK * CH
NR = 10240
RPT = NR // NS
HN = 10240
HPT = HN // NS

_mesh = plsc.VectorSubcoreMesh(core_axis_name="c", subcore_axis_name="s")


def _sc_body(with_deg, *refs):
    if with_deg:
        (x_hbm, src_hbm, dst_hbm, out0, out1, deg0, deg1,
         sb0, sb1, sb2, sb3, db0, db1, db2, db3,
         rw0, rw1, rw2, rw3, srct, dstt, rowst,
         onesb, sdeg, degstage, accum,
         gs0, gs1, gs2, gs3, ss0, ss1, ss2, ss3,
         is0, is1, is2, is3, tsem) = refs
    else:
        (x_hbm, src_hbm, dst_hbm, out0, out1,
         sb0, sb1, sb2, sb3, db0, db1, db2, db3,
         rw0, rw1, rw2, rw3, srct, dstt, rowst, accum,
         gs0, gs1, gs2, gs3, ss0, ss1, ss2, ss3,
         is0, is1, is2, is3, tsem) = refs
    srcb = (sb0, sb1, sb2, sb3)
    dstb = (db0, db1, db2, db3)
    rows = (rw0, rw1, rw2, rw3)
    gs = (gs0, gs1, gs2, gs3)
    ss = (ss0, ss1, ss2, ss3)
    isem = (is0, is1, is2, is3)

    cid = lax.axis_index("c")
    sid = lax.axis_index("s")
    wid = sid * NC + cid
    row0 = sid * RPT

    def _zb(i, c):
        for j in range(D // 16):
            rw0[i, pl.ds(j * 16, 16)] = jnp.zeros((16,), jnp.float32)
        return c
    lax.fori_loop(0, CH, _zb, 0)
    for k in range(RPT // CH):
        pltpu.sync_copy(rw0, accum.at[pl.ds(row0 + k * CH, CH), :])
    if with_deg:
        for i in range(CH // 16):
            onesb[pl.ds(i * 16, 16)] = jnp.ones((16,), jnp.float32)

        def _zd(i, c):
            degstage[pl.ds(i * 16, 16)] = jnp.zeros((16,), jnp.float32)
            return c
        lax.fori_loop(0, HPT // 16, _zd, 0)
        pltpu.sync_copy(degstage, sdeg.at[pl.ds(sid * HPT, HPT)])
    plsc.subcore_barrier()

    base = wid * EPW

    def _load(off, j):
        pltpu.async_copy(src_hbm.at[pl.ds(off, CH)], srcb[j], isem[j])
        pltpu.async_copy(dst_hbm.at[pl.ds(off, CH)], dstb[j], isem[j])

    def _wait_load(j):
        pltpu.make_async_copy(src_hbm.at[pl.ds(0, CH)], srcb[j], isem[j]).wait()
        pltpu.make_async_copy(dst_hbm.at[pl.ds(0, CH)], dstb[j], isem[j]).wait()

    def _gather(j):
        pltpu.async_copy(x_hbm.at[srcb[j]], rows[j], gs[j])

    def _wait_gather(j):
        pltpu.make_async_copy(x_hbm.at[srcb[j]], rows[j], gs[j]).wait()

    def _scatter(j):
        pltpu.async_copy(rows[j], accum.at[dstb[j]], ss[j], add=True)
        if with_deg:
            pltpu.async_copy(onesb, sdeg.at[dstb[j]], ss[j], add=True)

    def _wait_scatter(j):
        pltpu.make_async_copy(rows[j], accum.at[dstb[j]], ss[j]).wait()
        if with_deg:
            pltpu.make_async_copy(onesb, sdeg.at[dstb[j]], ss[j]).wait()

    _load(base, 0)
    _wait_load(0)
    _gather(0)
    _load(base + CH, 1)
    _wait_load(1)
    _gather(1)

    for b in range(NB):
        j2 = (b + 2) % NB
        if b >= 2:
            _wait_scatter(j2)
        _load(base + (b + 2) * CH, j2)
        _wait_gather(b)
        _scatter(b)
        _wait_load(j2)
        _gather(j2)

    def _block(g, c):
        for b in range(NB):
            ch = g * NB + b
            j2 = (b + 2) % NB

            @pl.when(ch + 2 < NCHUNK)
            def _():
                _wait_scatter(j2)
                _load(base + (ch + 2) * CH, j2)
            _wait_gather(b)
            _scatter(b)

            @pl.when(ch + 2 < NCHUNK)
            def _():
                _wait_load(j2)
                _gather(j2)
        return c
    lax.fori_loop(1, NCHUNK // NB, _block, 0)

    for j in range(NB):
        _wait_scatter(j)

    offt = base + NCHUNK * CH
    pltpu.sync_copy(src_hbm.at[pl.ds(offt, TAIL)], srct)
    pltpu.sync_copy(dst_hbm.at[pl.ds(offt, TAIL)], dstt)
    pltpu.async_copy(x_hbm.at[srct], rowst, tsem).wait()
    pltpu.sync_copy(rowst, accum.at[dstt], add=True)
    if with_deg:
        pltpu.sync_copy(onesb.at[pl.ds(0, TAIL)], sdeg.at[dstt], add=True)
    plsc.subcore_barrier()

    for k in range(RPT // CH):
        r0 = row0 + k * CH
        pltpu.sync_copy(accum.at[pl.ds(r0, CH), :], rw0)

        @pl.when(cid == 0)
        def _():
            pltpu.sync_copy(rw0, out0.at[pl.ds(r0, CH), :])

        @pl.when(cid == 1)
        def _():
            pltpu.sync_copy(rw0, out1.at[pl.ds(r0, CH), :])
    if with_deg:
        d0row = sid * HPT
        pltpu.sync_copy(sdeg.at[pl.ds(d0row, HPT)], degstage)

        @pl.when(cid == 0)
        def _():
            pltpu.sync_copy(degstage, deg0.at[pl.ds(d0row, HPT)])

        @pl.when(cid == 1)
        def _():
            pltpu.sync_copy(degstage, deg1.at[pl.ds(d0row, HPT)])


_f32 = jnp.float32


def _sc_scratch(with_deg):
    s = []
    s += [pltpu.VMEM((CH,), jnp.int32)] * 4
    s += [pltpu.VMEM((CH,), jnp.int32)] * 4
    s += [pltpu.VMEM((CH, D), _f32)] * 4
    s += [pltpu.VMEM((TAIL,), jnp.int32)] * 2
    s += [pltpu.VMEM((TAIL, D), _f32)]
    if with_deg:
        s += [pltpu.VMEM((CH,), _f32)]
        s += [pltpu.VMEM_SHARED((HN,), _f32)]
        s += [pltpu.VMEM((HPT,), _f32)]
    s += [pltpu.VMEM_SHARED((NR, D), _f32)]
    s += [pltpu.SemaphoreType.DMA] * 13
    return s


_sc_pass1 = pl.kernel(
    functools.partial(_sc_body, True),
    out_type=(jax.ShapeDtypeStruct((NR, D), _f32),
              jax.ShapeDtypeStruct((NR, D), _f32),
              jax.ShapeDtypeStruct((HN,), _f32),
              jax.ShapeDtypeStruct((HN,), _f32)),
    mesh=_mesh,
    scratch_types=_sc_scratch(True),
    name="sage_sc_pass1",
)

_sc_pass2 = pl.kernel(
    functools.partial(_sc_body, False),
    out_type=(jax.ShapeDtypeStruct((NR, D), _f32),
              jax.ShapeDtypeStruct((NR, D), _f32)),
    mesh=_mesh,
    scratch_types=_sc_scratch(False),
    name="sage_sc_pass2",
)

BM = 400
GRID = N // BM

_DN = (((1,), (1,)), ((), ()))


def _tc_mid_body(p0, p1, d0, d1, x, w1l, b1, w1r, w2l, b2, w2r, y2, z2):
    r = 1.0 / jnp.maximum(d0[...] + d1[...], 1.0)
    agg = (p0[...] + p1[...]) * r
    h = lax.dot_general(agg, w1l[...], _DN, preferred_element_type=_f32)
    h += lax.dot_general(x[...], w1r[...], _DN, preferred_element_type=_f32)
    h = jnp.maximum(h + b1[...], 0.0)
    y2[...] = lax.dot_general(h, w2l[...], _DN, preferred_element_type=_f32)
    z2[...] = lax.dot_general(h, w2r[...], _DN, preferred_element_type=_f32) + b2[...]


def _row_spec(c):
    return pl.BlockSpec((BM, c), lambda i: (i, 0))


def _full_spec(r, c):
    return pl.BlockSpec((r, c), lambda i: (0, 0))


_tc_mid = pl.pallas_call(
    _tc_mid_body,
    grid=(GRID,),
    in_specs=[
        _row_spec(D), _row_spec(D), _row_spec(1), _row_spec(1), _row_spec(D),
        _full_spec(HID, D), _full_spec(1, HID), _full_spec(HID, D),
        _full_spec(D, HID), _full_spec(1, D), _full_spec(D, HID),
    ],
    out_specs=[_row_spec(D), _row_spec(D)],
    out_shape=[jax.ShapeDtypeStruct((N, D), _f32),
               jax.ShapeDtypeStruct((N, D), _f32)],
    name="sage_tc_mid",
)


def _tc_out_body(p0, p1, d0, d1, z2, out):
    r = 1.0 / jnp.maximum(d0[...] + d1[...], 1.0)
    out[...] = (p0[...] + p1[...]) * r + z2[...]


_tc_out = pl.pallas_call(
    _tc_out_body,
    grid=(GRID,),
    in_specs=[_row_spec(D), _row_spec(D), _row_spec(1), _row_spec(1),
              _row_spec(D)],
    out_specs=_row_spec(D),
    out_shape=jax.ShapeDtypeStruct((N, D), _f32),
    name="sage_tc_out",
)


def kernel(x, edge_index, W1l, b1l, W1r, W2l, b2l, W2r):
    src = edge_index[0]
    dst = edge_index[1]
    p0, p1, dg0, dg1 = _sc_pass1(x, src, dst)
    d0 = dg0.reshape(HN, 1)
    d1 = dg1.reshape(HN, 1)
    y2, z2 = _tc_mid(p0, p1, d0, d1, x, W1l, b1l.reshape(1, HID), W1r,
                     W2l, b2l.reshape(1, D), W2r)
    q0, q1 = _sc_pass2(y2, src, dst)
    return _tc_out(q0, q1, d0, d1, z2)


# speedup vs baseline: 1.0318x; 1.0171x over previous
import functools

import jax
import jax.numpy as jnp
from jax import lax
from jax.experimental import pallas as pl
from jax.experimental.pallas import tpu as pltpu
from jax.experimental.pallas import tpu_sc as plsc

N = 10000
E = 320000
D = 128
HID = 256

NC, NS = 2, 16
NW = NC * NS
EPW = E // NW
CH = 64
NB = 4
NCHUNK = EPW // CH
TAIL = EPW - NCHUNK * CH
NR = 10240
RPT = NR // NS
HN = 10240
HPT = HN // NS

_mesh = plsc.VectorSubcoreMesh(core_axis_name="c", subcore_axis_name="s")


def _sc_body(with_deg, *refs):
    if with_deg:
        (x_hbm, src_hbm, dst_hbm, out0, out1, deg0, deg1,
         sb0, sb1, sb2, sb3, db0, db1, db2, db3,
         rw0, rw1, rw2, rw3, srct, dstt, rowst,
         onesb, sdeg, degstage, accum,
         gs0, gs1, gs2, gs3, ss0, ss1, ss2, ss3,
         is0, is1, is2, is3, tsem) = refs
    else:
        (x_hbm, src_hbm, dst_hbm, out0, out1,
         sb0, sb1, sb2, sb3, db0, db1, db2, db3,
         rw0, rw1, rw2, rw3, srct, dstt, rowst, accum,
         gs0, gs1, gs2, gs3, ss0, ss1, ss2, ss3,
         is0, is1, is2, is3, tsem) = refs
    srcb = (sb0, sb1, sb2, sb3)
    dstb = (db0, db1, db2, db3)
    rows = (rw0, rw1, rw2, rw3)
    gs = (gs0, gs1, gs2, gs3)
    ss = (ss0, ss1, ss2, ss3)
    isem = (is0, is1, is2, is3)

    cid = lax.axis_index("c")
    sid = lax.axis_index("s")
    wid = sid * NC + cid
    row0 = sid * RPT

    def _zb(i, c):
        for j in range(D // 16):
            rw0[i, pl.ds(j * 16, 16)] = jnp.zeros((16,), jnp.float32)
        return c
    lax.fori_loop(0, CH, _zb, 0)
    for k in range(RPT // CH):
        pltpu.sync_copy(rw0, accum.at[pl.ds(row0 + k * CH, CH), :])
    if with_deg:
        for i in range(CH // 16):
            onesb[pl.ds(i * 16, 16)] = jnp.ones((16,), jnp.float32)

        def _zd(i, c):
            degstage[pl.ds(i * 16, 16)] = jnp.zeros((16,), jnp.float32)
            return c
        lax.fori_loop(0, HPT // 16, _zd, 0)
        pltpu.sync_copy(degstage, sdeg.at[pl.ds(sid * HPT, HPT)])
    plsc.subcore_barrier()

    base = wid * EPW

    def _load(off, j):
        pltpu.async_copy(src_hbm.at[pl.ds(off, CH)], srcb[j], isem[j])
        pltpu.async_copy(dst_hbm.at[pl.ds(off, CH)], dstb[j], isem[j])

    def _wait_load(j):
        pltpu.make_async_copy(src_hbm.at[pl.ds(0, CH)], srcb[j], isem[j]).wait()
        pltpu.make_async_copy(dst_hbm.at[pl.ds(0, CH)], dstb[j], isem[j]).wait()

    def _gather(j):
        pltpu.async_copy(x_hbm.at[srcb[j]], rows[j], gs[j])

    def _wait_gather(j):
        pltpu.make_async_copy(x_hbm.at[srcb[j]], rows[j], gs[j]).wait()

    def _scatter(j):
        pltpu.async_copy(rows[j], accum.at[dstb[j]], ss[j], add=True)
        if with_deg:
            pltpu.async_copy(onesb, sdeg.at[dstb[j]], ss[j], add=True)

    def _wait_scatter(j):
        pltpu.make_async_copy(rows[j], accum.at[dstb[j]], ss[j]).wait()
        if with_deg:
            pltpu.make_async_copy(onesb, sdeg.at[dstb[j]], ss[j]).wait()

    _load(base, 0)
    _wait_load(0)
    _gather(0)
    _load(base + CH, 1)
    _wait_load(1)
    _gather(1)

    for b in range(NB):
        j2 = (b + 2) % NB
        if b >= 2:
            _wait_scatter(j2)
        _load(base + (b + 2) * CH, j2)
        _wait_gather(b)
        _scatter(b)
        _wait_load(j2)
        _gather(j2)

    def _block(g, c):
        for b in range(NB):
            ch = g * NB + b
            j2 = (b + 2) % NB

            @pl.when(ch + 2 < NCHUNK)
            def _():
                _wait_scatter(j2)
                _load(base + (ch + 2) * CH, j2)
            _wait_gather(b)
            _scatter(b)

            @pl.when(ch + 2 < NCHUNK)
            def _():
                _wait_load(j2)
                _gather(j2)
        return c
    lax.fori_loop(1, NCHUNK // NB, _block, 0)

    for j in range(NB):
        _wait_scatter(j)

    offt = base + NCHUNK * CH
    pltpu.sync_copy(src_hbm.at[pl.ds(offt, TAIL)], srct)
    pltpu.sync_copy(dst_hbm.at[pl.ds(offt, TAIL)], dstt)
    pltpu.async_copy(x_hbm.at[srct], rowst, tsem).wait()
    pltpu.sync_copy(rowst, accum.at[dstt], add=True)
    if with_deg:
        pltpu.sync_copy(onesb.at[pl.ds(0, TAIL)], sdeg.at[dstt], add=True)
    plsc.subcore_barrier()

    stg = (rw0, rw1)

    def _stage_in(k, j):
        pltpu.async_copy(accum.at[pl.ds(row0 + k * CH, CH), :], stg[j], gs[j])

    def _wait_in(k, j):
        pltpu.make_async_copy(accum.at[pl.ds(row0 + k * CH, CH), :], stg[j],
                              gs[j]).wait()

    def _stage_out(k, j):
        r0 = row0 + k * CH

        @pl.when(cid == 0)
        def _():
            pltpu.async_copy(stg[j], out0.at[pl.ds(r0, CH), :], ss[j])

        @pl.when(cid == 1)
        def _():
            pltpu.async_copy(stg[j], out1.at[pl.ds(r0, CH), :], ss[j])

    def _wait_out(k, j):
        r0 = row0 + k * CH

        @pl.when(cid == 0)
        def _():
            pltpu.make_async_copy(stg[j], out0.at[pl.ds(r0, CH), :],
                                  ss[j]).wait()

        @pl.when(cid == 1)
        def _():
            pltpu.make_async_copy(stg[j], out1.at[pl.ds(r0, CH), :],
                                  ss[j]).wait()

    NK = RPT // CH
    _stage_in(0, 0)
    for k in range(NK):
        j = k % 2
        if k + 1 < NK:
            if k >= 1:
                _wait_out(k - 1, 1 - j)
            _stage_in(k + 1, 1 - j)
        _wait_in(k, j)
        _stage_out(k, j)
    _wait_out(NK - 2, (NK - 2) % 2)
    _wait_out(NK - 1, (NK - 1) % 2)
    if with_deg:
        d0row = sid * HPT
        pltpu.sync_copy(sdeg.at[pl.ds(d0row, HPT)], degstage)

        @pl.when(cid == 0)
        def _():
            pltpu.sync_copy(degstage, deg0.at[pl.ds(d0row, HPT)])

        @pl.when(cid == 1)
        def _():
            pltpu.sync_copy(degstage, deg1.at[pl.ds(d0row, HPT)])


_f32 = jnp.float32


def _sc_scratch(with_deg):
    s = []
    s += [pltpu.VMEM((CH,), jnp.int32)] * 4
    s += [pltpu.VMEM((CH,), jnp.int32)] * 4
    s += [pltpu.VMEM((CH, D), _f32)] * 4
    s += [pltpu.VMEM((TAIL,), jnp.int32)] * 2
    s += [pltpu.VMEM((TAIL, D), _f32)]
    if with_deg:
        s += [pltpu.VMEM((CH,), _f32)]
        s += [pltpu.VMEM_SHARED((HN,), _f32)]
        s += [pltpu.VMEM((HPT,), _f32)]
    s += [pltpu.VMEM_SHARED((NR, D), _f32)]
    s += [pltpu.SemaphoreType.DMA] * 13
    return s


_sc_pass1 = pl.kernel(
    functools.partial(_sc_body, True),
    out_type=(jax.ShapeDtypeStruct((NR, D), _f32),
              jax.ShapeDtypeStruct((NR, D), _f32),
              jax.ShapeDtypeStruct((HN,), _f32),
              jax.ShapeDtypeStruct((HN,), _f32)),
    mesh=_mesh,
    scratch_types=_sc_scratch(True),
    name="sage_sc_pass1",
)

_sc_pass2 = pl.kernel(
    functools.partial(_sc_body, False),
    out_type=(jax.ShapeDtypeStruct((NR, D), _f32),
              jax.ShapeDtypeStruct((NR, D), _f32)),
    mesh=_mesh,
    scratch_types=_sc_scratch(False),
    name="sage_sc_pass2",
)

BM = 400
GRID = N // BM

_DN = (((1,), (1,)), ((), ()))


def _tc_mid_body(p0, p1, d0, d1, x, w1l, b1, w1r, w2l, b2, w2r, y2, z2):
    r = 1.0 / jnp.maximum(d0[...] + d1[...], 1.0)
    agg = (p0[...] + p1[...]) * r
    h = lax.dot_general(agg, w1l[...], _DN, preferred_element_type=_f32)
    h += lax.dot_general(x[...], w1r[...], _DN, preferred_element_type=_f32)
    h = jnp.maximum(h + b1[...], 0.0)
    y2[...] = lax.dot_general(h, w2l[...], _DN, preferred_element_type=_f32)
    z2[...] = lax.dot_general(h, w2r[...], _DN, preferred_element_type=_f32) + b2[...]


def _row_spec(c):
    return pl.BlockSpec((BM, c), lambda i: (i, 0))


def _full_spec(r, c):
    return pl.BlockSpec((r, c), lambda i: (0, 0))


_tc_mid = pl.pallas_call(
    _tc_mid_body,
    grid=(GRID,),
    in_specs=[
        _row_spec(D), _row_spec(D), _row_spec(1), _row_spec(1), _row_spec(D),
        _full_spec(HID, D), _full_spec(1, HID), _full_spec(HID, D),
        _full_spec(D, HID), _full_spec(1, D), _full_spec(D, HID),
    ],
    out_specs=[_row_spec(D), _row_spec(D)],
    out_shape=[jax.ShapeDtypeStruct((N, D), _f32),
               jax.ShapeDtypeStruct((N, D), _f32)],
    name="sage_tc_mid",
)


def _tc_out_body(p0, p1, d0, d1, z2, out):
    r = 1.0 / jnp.maximum(d0[...] + d1[...], 1.0)
    out[...] = (p0[...] + p1[...]) * r + z2[...]


_tc_out = pl.pallas_call(
    _tc_out_body,
    grid=(GRID,),
    in_specs=[_row_spec(D), _row_spec(D), _row_spec(1), _row_spec(1),
              _row_spec(D)],
    out_specs=_row_spec(D),
    out_shape=jax.ShapeDtypeStruct((N, D), _f32),
    name="sage_tc_out",
)


def kernel(x, edge_index, W1l, b1l, W1r, W2l, b2l, W2r):
    src = edge_index[0]
    dst = edge_index[1]
    p0, p1, dg0, dg1 = _sc_pass1(x, src, dst)
    d0 = dg0.reshape(HN, 1)
    d1 = dg1.reshape(HN, 1)
    y2, z2 = _tc_mid(p0, p1, d0, d1, x, W1l, b1l.reshape(1, HID), W1r,
                     W2l, b2l.reshape(1, D), W2r)
    q0, q1 = _sc_pass2(y2, src, dst)
    return _tc_out(q0, q1, d0, d1, z2)


# speedup vs baseline: 1.0353x; 1.0033x over previous
import functools

import jax
import jax.numpy as jnp
from jax import lax
from jax.experimental import pallas as pl
from jax.experimental.pallas import tpu as pltpu
from jax.experimental.pallas import tpu_sc as plsc

N = 10000
E = 320000
D = 128
HID = 256

NC, NS = 2, 16
NW = NC * NS
EPW = E // NW
CH = 64
NB = 4
NCHUNK = EPW // CH
TAIL = EPW - NCHUNK * CH
NR = 10240
RPT = NR // NS
HN = 10240
HPT = HN // NS

_mesh = plsc.VectorSubcoreMesh(core_axis_name="c", subcore_axis_name="s")


def _sc_body(with_deg, *refs):
    if with_deg:
        (x_hbm, src_hbm, dst_hbm, out0, out1, deg0, deg1,
         sb0, sb1, sb2, sb3, db0, db1, db2, db3,
         rw0, rw1, rw2, rw3, srct, dstt, rowst,
         onesb, sdeg, degstage, accum,
         gs0, gs1, gs2, gs3, ss0, ss1, ss2, ss3,
         is0, is1, is2, is3, tsem) = refs
    else:
        (x_hbm, src_hbm, dst_hbm, out0, out1,
         sb0, sb1, sb2, sb3, db0, db1, db2, db3,
         rw0, rw1, rw2, rw3, srct, dstt, rowst, accum,
         gs0, gs1, gs2, gs3, ss0, ss1, ss2, ss3,
         is0, is1, is2, is3, tsem) = refs
    srcb = (sb0, sb1, sb2, sb3)
    dstb = (db0, db1, db2, db3)
    rows = (rw0, rw1, rw2, rw3)
    gs = (gs0, gs1, gs2, gs3)
    ss = (ss0, ss1, ss2, ss3)
    isem = (is0, is1, is2, is3)

    cid = lax.axis_index("c")
    sid = lax.axis_index("s")
    wid = sid * NC + cid
    row0 = sid * RPT

    def _zb(i, c):
        for j in range(D // 16):
            rw0[i, pl.ds(j * 16, 16)] = jnp.zeros((16,), jnp.float32)
        return c
    lax.fori_loop(0, CH, _zb, 0)
    for k in range(RPT // CH):
        if k >= 2:
            pltpu.make_async_copy(
                rw0, accum.at[pl.ds(row0 + (k - 2) * CH, CH), :],
                tsem).wait()
        pltpu.async_copy(rw0, accum.at[pl.ds(row0 + k * CH, CH), :], tsem)
    for k in range(RPT // CH - 2, RPT // CH):
        pltpu.make_async_copy(rw0, accum.at[pl.ds(row0 + k * CH, CH), :],
                              tsem).wait()
    if with_deg:
        for i in range(CH // 16):
            onesb[pl.ds(i * 16, 16)] = jnp.ones((16,), jnp.float32)

        def _zd(i, c):
            degstage[pl.ds(i * 16, 16)] = jnp.zeros((16,), jnp.float32)
            return c
        lax.fori_loop(0, HPT // 16, _zd, 0)
        pltpu.sync_copy(degstage, sdeg.at[pl.ds(sid * HPT, HPT)])
    plsc.subcore_barrier()

    base = wid * EPW

    def _load(off, j):
        pltpu.async_copy(src_hbm.at[pl.ds(off, CH)], srcb[j], isem[j])
        pltpu.async_copy(dst_hbm.at[pl.ds(off, CH)], dstb[j], isem[j])

    def _wait_load(j):
        pltpu.make_async_copy(src_hbm.at[pl.ds(0, CH)], srcb[j], isem[j]).wait()
        pltpu.make_async_copy(dst_hbm.at[pl.ds(0, CH)], dstb[j], isem[j]).wait()

    def _gather(j):
        pltpu.async_copy(x_hbm.at[srcb[j]], rows[j], gs[j])

    def _wait_gather(j):
        pltpu.make_async_copy(x_hbm.at[srcb[j]], rows[j], gs[j]).wait()

    def _scatter(j):
        pltpu.async_copy(rows[j], accum.at[dstb[j]], ss[j], add=True)
        if with_deg:
            pltpu.async_copy(onesb, sdeg.at[dstb[j]], ss[j], add=True)

    def _wait_scatter(j):
        pltpu.make_async_copy(rows[j], accum.at[dstb[j]], ss[j]).wait()
        if with_deg:
            pltpu.make_async_copy(onesb, sdeg.at[dstb[j]], ss[j]).wait()

    _load(base, 0)
    _wait_load(0)
    _gather(0)
    _load(base + CH, 1)
    _wait_load(1)
    _gather(1)

    for b in range(NB):
        j2 = (b + 2) % NB
        if b >= 2:
            _wait_scatter(j2)
        _load(base + (b + 2) * CH, j2)
        _wait_gather(b)
        _scatter(b)
        _wait_load(j2)
        _gather(j2)

    def _block(g, c):
        for b in range(NB):
            ch = g * NB + b
            j2 = (b + 2) % NB

            @pl.when(ch + 2 < NCHUNK)
            def _():
                _wait_scatter(j2)
                _load(base + (ch + 2) * CH, j2)
            _wait_gather(b)
            _scatter(b)

            @pl.when(ch + 2 < NCHUNK)
            def _():
                _wait_load(j2)
                _gather(j2)
        return c
    lax.fori_loop(1, NCHUNK // NB, _block, 0)

    for j in range(NB):
        _wait_scatter(j)

    offt = base + NCHUNK * CH
    pltpu.sync_copy(src_hbm.at[pl.ds(offt, TAIL)], srct)
    pltpu.sync_copy(dst_hbm.at[pl.ds(offt, TAIL)], dstt)
    pltpu.async_copy(x_hbm.at[srct], rowst, tsem).wait()
    pltpu.sync_copy(rowst, accum.at[dstt], add=True)
    if with_deg:
        pltpu.sync_copy(onesb.at[pl.ds(0, TAIL)], sdeg.at[dstt], add=True)
    plsc.subcore_barrier()

    stg = (rw0, rw1)

    def _stage_in(k, j):
        pltpu.async_copy(accum.at[pl.ds(row0 + k * CH, CH), :], stg[j], gs[j])

    def _wait_in(k, j):
        pltpu.make_async_copy(accum.at[pl.ds(row0 + k * CH, CH), :], stg[j],
                              gs[j]).wait()

    def _stage_out(k, j):
        r0 = row0 + k * CH

        @pl.when(cid == 0)
        def _():
            pltpu.async_copy(stg[j], out0.at[pl.ds(r0, CH), :], ss[j])

        @pl.when(cid == 1)
        def _():
            pltpu.async_copy(stg[j], out1.at[pl.ds(r0, CH), :], ss[j])

    def _wait_out(k, j):
        r0 = row0 + k * CH

        @pl.when(cid == 0)
        def _():
            pltpu.make_async_copy(stg[j], out0.at[pl.ds(r0, CH), :],
                                  ss[j]).wait()

        @pl.when(cid == 1)
        def _():
            pltpu.make_async_copy(stg[j], out1.at[pl.ds(r0, CH), :],
                                  ss[j]).wait()

    NK = RPT // CH
    _stage_in(0, 0)
    for k in range(NK):
        j = k % 2
        if k + 1 < NK:
            if k >= 1:
                _wait_out(k - 1, 1 - j)
            _stage_in(k + 1, 1 - j)
        _wait_in(k, j)
        _stage_out(k, j)
    _wait_out(NK - 2, (NK - 2) % 2)
    _wait_out(NK - 1, (NK - 1) % 2)
    if with_deg:
        d0row = sid * HPT
        pltpu.sync_copy(sdeg.at[pl.ds(d0row, HPT)], degstage)

        @pl.when(cid == 0)
        def _():
            pltpu.sync_copy(degstage, deg0.at[pl.ds(d0row, HPT)])

        @pl.when(cid == 1)
        def _():
            pltpu.sync_copy(degstage, deg1.at[pl.ds(d0row, HPT)])


_f32 = jnp.float32


def _sc_scratch(with_deg):
    s = []
    s += [pltpu.VMEM((CH,), jnp.int32)] * 4
    s += [pltpu.VMEM((CH,), jnp.int32)] * 4
    s += [pltpu.VMEM((CH, D), _f32)] * 4
    s += [pltpu.VMEM((TAIL,), jnp.int32)] * 2
    s += [pltpu.VMEM((TAIL, D), _f32)]
    if with_deg:
        s += [pltpu.VMEM((CH,), _f32)]
        s += [pltpu.VMEM_SHARED((HN,), _f32)]
        s += [pltpu.VMEM((HPT,), _f32)]
    s += [pltpu.VMEM_SHARED((NR, D), _f32)]
    s += [pltpu.SemaphoreType.DMA] * 13
    return s


_sc_pass1 = pl.kernel(
    functools.partial(_sc_body, True),
    out_type=(jax.ShapeDtypeStruct((NR, D), _f32),
              jax.ShapeDtypeStruct((NR, D), _f32),
              jax.ShapeDtypeStruct((HN,), _f32),
              jax.ShapeDtypeStruct((HN,), _f32)),
    mesh=_mesh,
    scratch_types=_sc_scratch(True),
    name="sage_sc_pass1",
)

_sc_pass2 = pl.kernel(
    functools.partial(_sc_body, False),
    out_type=(jax.ShapeDtypeStruct((NR, D), _f32),
              jax.ShapeDtypeStruct((NR, D), _f32)),
    mesh=_mesh,
    scratch_types=_sc_scratch(False),
    name="sage_sc_pass2",
)

BM = 400
GRID = N // BM

_DN = (((1,), (1,)), ((), ()))


def _tc_mid_body(p0, p1, d0, d1, x, w1l, b1, w1r, w2l, b2, w2r, y2, z2):
    r = 1.0 / jnp.maximum(d0[...] + d1[...], 1.0)
    agg = (p0[...] + p1[...]) * r
    h = lax.dot_general(agg, w1l[...], _DN, preferred_element_type=_f32)
    h += lax.dot_general(x[...], w1r[...], _DN, preferred_element_type=_f32)
    h = jnp.maximum(h + b1[...], 0.0)
    y2[...] = lax.dot_general(h, w2l[...], _DN, preferred_element_type=_f32)
    z2[...] = lax.dot_general(h, w2r[...], _DN, preferred_element_type=_f32) + b2[...]


def _row_spec(c):
    return pl.BlockSpec((BM, c), lambda i: (i, 0))


def _full_spec(r, c):
    return pl.BlockSpec((r, c), lambda i: (0, 0))


_tc_mid = pl.pallas_call(
    _tc_mid_body,
    grid=(GRID,),
    in_specs=[
        _row_spec(D), _row_spec(D), _row_spec(1), _row_spec(1), _row_spec(D),
        _full_spec(HID, D), _full_spec(1, HID), _full_spec(HID, D),
        _full_spec(D, HID), _full_spec(1, D), _full_spec(D, HID),
    ],
    out_specs=[_row_spec(D), _row_spec(D)],
    out_shape=[jax.ShapeDtypeStruct((N, D), _f32),
               jax.ShapeDtypeStruct((N, D), _f32)],
    name="sage_tc_mid",
)


def _tc_out_body(p0, p1, d0, d1, z2, out):
    r = 1.0 / jnp.maximum(d0[...] + d1[...], 1.0)
    out[...] = (p0[...] + p1[...]) * r + z2[...]


_tc_out = pl.pallas_call(
    _tc_out_body,
    grid=(GRID,),
    in_specs=[_row_spec(D), _row_spec(D), _row_spec(1), _row_spec(1),
              _row_spec(D)],
    out_specs=_row_spec(D),
    out_shape=jax.ShapeDtypeStruct((N, D), _f32),
    name="sage_tc_out",
)


def kernel(x, edge_index, W1l, b1l, W1r, W2l, b2l, W2r):
    src = edge_index[0]
    dst = edge_index[1]
    p0, p1, dg0, dg1 = _sc_pass1(x, src, dst)
    d0 = dg0.reshape(HN, 1)
    d1 = dg1.reshape(HN, 1)
    y2, z2 = _tc_mid(p0, p1, d0, d1, x, W1l, b1l.reshape(1, HID), W1r,
                     W2l, b2l.reshape(1, D), W2r)
    q0, q1 = _sc_pass2(y2, src, dst)
    return _tc_out(q0, q1, d0, d1, z2)
